# baseline (speedup 1.0000x reference)
import functools

import jax
import jax.numpy as jnp
from jax import lax
from jax.experimental import pallas as pl
from jax.experimental.pallas import tpu as pltpu
from jax.experimental.pallas import tpu_sc as plsc

N = 16384
E = 262144
BLK = 1024
GRID = N // BLK
CH = 64
NBUF = 8
NSUB = 16
NCORE = 2
NW = NSUB * NCORE
EPW = E // NW
NCH = EPW // CH
RPW = N // NSUB
ZR = 128

_f32 = jnp.float32


W64 = 64


@functools.lru_cache(maxsize=None)
def _make_segsum(T):
    mesh = plsc.VectorSubcoreMesh(core_axis_name="c", subcore_axis_name="s")
    out_type = tuple(jax.ShapeDtypeStruct((2 * N, W64), _f32)
                     for _ in range(T))
    scratch = [pltpu.VMEM((NCH, CH), jnp.int32),
               pltpu.VMEM((NCH, CH), jnp.int32)]
    scratch += [pltpu.VMEM((CH, W64), _f32)] * NBUF
    scratch += [pltpu.VMEM((ZR, W64), _f32),
                pltpu.VMEM_SHARED((N, W64), _f32)]
    scratch += [pltpu.SemaphoreType.DMA] * NBUF

    def body(src_r, dst_r, *rest):
        tabs = rest[:T]
        outs = rest[T:2 * T]
        sidx, didx = rest[2 * T], rest[2 * T + 1]
        bufs = rest[2 * T + 2:2 * T + 2 + NBUF]
        zbuf, acc = rest[2 * T + 2 + NBUF], rest[2 * T + 3 + NBUF]
        sems = rest[2 * T + 4 + NBUF:]

        c = lax.axis_index("c")
        s = lax.axis_index("s")
        wid = c * NSUB + s
        base_r = s * RPW

        pltpu.sync_copy(src_r.at[pl.ds(wid * NCH, NCH)], sidx)
        pltpu.sync_copy(dst_r.at[pl.ds(wid * NCH, NCH)], didx)

        def zinit(i, _):
            zbuf[i // 4, pl.ds((i % 4) * 16, 16)] = jnp.zeros((16,), _f32)
            return 0

        lax.fori_loop(0, ZR * 4, zinit, 0)
        for r0 in range(0, RPW, ZR):
            pltpu.sync_copy(zbuf, acc.at[pl.ds(base_r + r0, ZR)])
        plsc.subcore_barrier()

        for t in range(T):
            tab, out = tabs[t], outs[t]

            def start(i, b, tab=tab):
                pltpu.async_copy(tab.at[sidx.at[i]], bufs[b], sems[b])

            def drain_scatter(i, b, tab=tab):
                pltpu.make_async_copy(tab.at[sidx.at[i]], bufs[b],
                                      sems[b]).wait()
                pltpu.sync_copy(bufs[b], acc.at[didx.at[i]], add=True)

            for k in range(NBUF - 1):
                start(k, k)

            def ring(g, _):
                i = NBUF * g
                for k in range(NBUF):
                    drain_scatter(i + k, k)
                    start(i + k + NBUF - 1, (k + NBUF - 1) % NBUF)
                return 0

            lax.fori_loop(0, NCH // NBUF - 1, ring, 0)
            i = NCH - NBUF
            start(NCH - 1, NBUF - 1)
            for k in range(NBUF):
                drain_scatter(i + k, k)
            plsc.subcore_barrier()

            pltpu.sync_copy(acc.at[pl.ds(base_r, RPW)],
                            out.at[pl.ds(c * N + base_r, RPW)])
            if t + 1 < T:
                for r0 in range(0, RPW, ZR):
                    pltpu.sync_copy(zbuf, acc.at[pl.ds(base_r + r0, ZR)])
            plsc.subcore_barrier()

    return pl.kernel(body, out_type=out_type, mesh=mesh,
                     scratch_types=scratch,
                     compiler_params=pltpu.CompilerParams(
                         use_tc_tiling_on_sc=False))


def _edge_segsum(src, dst, tables, widths):
    del widths
    return _make_segsum(len(tables))(src, dst, *tables)


def _mm(x, w):
    return lax.dot_general(x, w, (((1,), (0,)), ((), ())),
                           preferred_element_type=_f32)


def _full(a):
    return pl.BlockSpec(a.shape, lambda i: (0,) * a.ndim)


def _rowblk(w, half=0):
    return pl.BlockSpec((BLK, w), lambda i, h=half: (i + h * GRID, 0))


def _leaky(x, s):
    return jnp.where(x >= 0, x, s * x)


def _rff(pos, encb):
    def body(pos_r, encb_r, xa_r, xb_r):
        vp = (2.0 * jnp.pi) * lax.dot_general(
            pos_r[...], encb_r[...], (((1,), (1,)), ((), ())),
            preferred_element_type=_f32)
        xa_r[...] = jnp.cos(vp)
        xb_r[...] = jnp.sin(vp)

    return pl.pallas_call(
        body, grid=(GRID,),
        in_specs=[_rowblk(3), _full(encb)],
        out_specs=[_rowblk(64), _rowblk(64)],
        out_shape=[jax.ShapeDtypeStruct((N, 64), _f32)] * 2,
    )(pos, encb)


def _conv(xa, xb, pos, sp, s0, s1, pp, with_max):
    h1w, h1b = pp['h1']
    h2w, h2b = pp['h2']
    g1w, g1b = pp['g1']
    g2w, g2b = pp['g2']
    g1pw, g1xw = g1w[:3], g1w[3:]
    wts = [h1w, h1b.reshape(1, -1), h2w, h2b.reshape(1, -1),
           g1pw, g1xw, g1b.reshape(1, -1), g2w, g2b.reshape(1, -1)]

    def body(xa_r, xb_r, pos_r, spa_r, spb_r, s0a_r, s0b_r, s1a_r, s1b_r,
             h1w_r, h1b_r, h2w_r, h2b_r, g1pw_r, g1xw_r, g1b_r, g2w_r,
             g2b_r, ya_r, yb_r, *mx_r):
        x = jnp.concatenate([xa_r[...], xb_r[...]], axis=1)
        t = jnp.maximum(_mm(x, h1w_r[...]) + h1b_r[...], 0.0)
        delta = jnp.tanh(_mm(t, h2w_r[...]) + h2b_r[...])
        spv = spa_r[...] + spb_r[...]
        apos = spv[:, 0:3]
        cnt = spv[:, 3:4]
        sx = jnp.concatenate([s0a_r[...] + s0b_r[...],
                              s1a_r[...] + s1b_r[...]], axis=1)
        inv = 1.0 / jnp.maximum(cnt, 1.0)
        mpos = (apos - cnt * pos_r[...] + cnt * delta) * inv
        mfeat = sx * inv
        o = jnp.maximum(_mm(mpos, g1pw_r[...]) + _mm(mfeat, g1xw_r[...])
                        + g1b_r[...], 0.0)
        o = jnp.maximum(_mm(o, g2w_r[...]) + g2b_r[...], 0.0)
        y = x + o
        ya_r[...] = y[:, :64]
        yb_r[...] = y[:, 64:]
        if mx_r:
            @pl.when(pl.program_id(0) == 0)
            def _():
                mx_r[0][...] = jnp.full((1, 128), -jnp.inf, _f32)
            mx_r[0][...] = jnp.maximum(mx_r[0][...],
                                       jnp.max(y, axis=0, keepdims=True))

    out_specs = [_rowblk(64), _rowblk(64)]
    out_shape = [jax.ShapeDtypeStruct((N, 64), _f32)] * 2
    if with_max:
        out_specs.append(pl.BlockSpec((1, 128), lambda i: (0, 0)))
        out_shape.append(jax.ShapeDtypeStruct((1, 128), _f32))
    return pl.pallas_call(
        body, grid=(GRID,),
        in_specs=[_rowblk(64), _rowblk(64), _rowblk(3),
                  _rowblk(64), _rowblk(64, 1),
                  _rowblk(64), _rowblk(64, 1), _rowblk(64), _rowblk(64, 1)]
                 + [_full(w) for w in wts],
        out_specs=out_specs, out_shape=out_shape,
    )(xa, xb, pos, sp, sp, s0, s0, s1, s1, *wts)


def _linkx_front(g, xm_src, pp, c1w_r, c1b_r, ndw_r, ndb_r, c2w_r, c2b_r,
                 f1w_r, f1b_r):
    out = g + _mm(g, c1w_r[...]) + c1b_r[...]
    xm = _mm(xm_src, ndw_r[...]) + ndb_r[...]
    out = out + xm + _mm(xm, c2w_r[...]) + c2b_r[...]
    out = jnp.maximum(out, 0.0)
    return jnp.maximum(_mm(out, f1w_r[...]) + f1b_r[...], 0.0)


def _bn_f2(h1, ssum_r, ssq_r, bnw_r, bnb_r, f2w_r, f2b_r):
    mu = ssum_r[...] * (1.0 / N)
    var = ssq_r[...] * (1.0 / N) - mu * mu
    scale = bnw_r[...] / jnp.sqrt(var + 1e-5)
    shift = bnb_r[...] - mu * scale
    return _mm(h1 * scale + shift, f2w_r[...]) + f2b_r[...]


def _stats(h1, ssum_r, ssq_r):
    @pl.when(pl.program_id(0) == 0)
    def _():
        ssum_r[...] = jnp.zeros((1, 256), _f32)
        ssq_r[...] = jnp.zeros((1, 256), _f32)
    ssum_r[...] += jnp.sum(h1, axis=0, keepdims=True)
    ssq_r[...] += jnp.sum(h1 * h1, axis=0, keepdims=True)


def _linkx_wts(pp):
    return [pp['edge'][1].reshape(1, -1), pp['cat1'][0],
            pp['cat1'][1].reshape(1, -1), pp['node'][0],
            pp['node'][1].reshape(1, -1), pp['cat2'][0],
            pp['cat2'][1].reshape(1, -1), pp['f1'][0],
            pp['f1'][1].reshape(1, -1)]


def _gcat(g0a, g0b, g1a, g1b, g2a, g2b, g3a, g3b, eb_r):
    return jnp.concatenate([g0a[...] + g0b[...], g1a[...] + g1b[...],
                            g2a[...] + g2b[...], g3a[...] + g3b[...]],
                           axis=1) + eb_r[...]


def _gspecs():
    return [_rowblk(64), _rowblk(64, 1)] * 4


_STAT = pl.BlockSpec((1, 256), lambda i: (0, 0))


def _tail_linkx1(xa, xb, mx, gparts, params):
    globw, globb = params['glob']
    t1w, t1b = params['tail1']
    t2w, t2b = params['tail2']
    wts = [globw, globb.reshape(1, -1), t1w[:128], t1w[128:],
           t1b.reshape(1, -1), t2w, t2b.reshape(1, -1)] \
        + _linkx_wts(params['linkx1'])

    def body(xa_r, xb_r, mx_r, g0a, g0b, g1a, g1b, g2a, g2b, g3a, g3b,
             gw_r, gb_r, t1x_r, t1h_r, t1b_r, t2w_r, t2b_r,
             eb_r, c1w_r, c1b_r, ndw_r, ndb_r, c2w_r, c2b_r, f1w_r, f1b_r,
             pc_r, xcat_r, h1_r, ssum_r, ssq_r):
        h = _leaky(_mm(mx_r[...], gw_r[...]) + gb_r[...], 0.2)
        x3 = jnp.concatenate([xa_r[...], xb_r[...]], axis=1)
        t = _leaky(_mm(x3, t1x_r[...]) + _mm(h, t1h_r[...]) + t1b_r[...], 0.2)
        pc_r[...] = jnp.tanh(_mm(t, t2w_r[...]) + t2b_r[...])
        xcat = jnp.concatenate([x3, jnp.broadcast_to(h, (BLK, 128))], axis=1)
        xcat_r[...] = xcat
        g = _gcat(g0a, g0b, g1a, g1b, g2a, g2b, g3a, g3b, eb_r)
        h1 = _linkx_front(g, xcat, None, c1w_r, c1b_r, ndw_r, ndb_r,
                          c2w_r, c2b_r, f1w_r, f1b_r)
        h1_r[...] = h1
        _stats(h1, ssum_r, ssq_r)

    return pl.pallas_call(
        body, grid=(GRID,),
        in_specs=[_rowblk(64), _rowblk(64),
                  pl.BlockSpec((1, 128), lambda i: (0, 0))] + _gspecs()
                 + [_full(w) for w in wts],
        out_specs=[_rowblk(3), _rowblk(256), _rowblk(256), _STAT, _STAT],
        out_shape=[jax.ShapeDtypeStruct((N, 3), _f32),
                   jax.ShapeDtypeStruct((N, 256), _f32),
                   jax.ShapeDtypeStruct((N, 256), _f32),
                   jax.ShapeDtypeStruct((1, 256), _f32),
                   jax.ShapeDtypeStruct((1, 256), _f32)],
    )(xa, xb, mx, gparts[0], gparts[0], gparts[1], gparts[1], gparts[2],
      gparts[2], gparts[3], gparts[3], *wts)


def _linkx12(h1, ssum, ssq, gparts, params):
    pp1 = params['linkx1']
    wts = [pp1['bn'][0].reshape(1, -1), pp1['bn'][1].reshape(1, -1),
           pp1['f2'][0], pp1['f2'][1].reshape(1, -1)] \
        + _linkx_wts(params['linkx2'])

    def body(h1_r, ssum_r, ssq_r, g0a, g0b, g1a, g1b, g2a, g2b, g3a, g3b,
             bnw_r, bnb_r, f2w_r, f2b_r,
             eb_r, c1w_r, c1b_r, ndw_r, ndb_r, c2w_r, c2b_r, f1w_r, f1b_r,
             h2_r, ssum2_r, ssq2_r):
        g1out = _bn_f2(h1_r[...], ssum_r, ssq_r, bnw_r, bnb_r, f2w_r, f2b_r)
        g = _gcat(g0a, g0b, g1a, g1b, g2a, g2b, g3a, g3b, eb_r)
        h2 = _linkx_front(g, g1out, None, c1w_r, c1b_r, ndw_r, ndb_r,
                          c2w_r, c2b_r, f1w_r, f1b_r)
        h2_r[...] = h2
        _stats(h2, ssum2_r, ssq2_r)

    return pl.pallas_call(
        body, grid=(GRID,),
        in_specs=[_rowblk(256), _STAT, _STAT] + _gspecs()
                 + [_full(w) for w in wts],
        out_specs=[_rowblk(256), _STAT, _STAT],
        out_shape=[jax.ShapeDtypeStruct((N, 256), _f32),
                   jax.ShapeDtypeStruct((1, 256), _f32),
                   jax.ShapeDtypeStruct((1, 256), _f32)],
    )(h1, ssum, ssq, gparts[0], gparts[0], gparts[1], gparts[1],
      gparts[2], gparts[2], gparts[3], gparts[3], *wts)


def _linkx2_heads(h2, ssum2, ssq2, xcat, pc, params):
    pp2 = params['linkx2']
    d1w, d1b = params['dec1']
    d2w, d2b = params['dec2']
    hd = params['heads']
    hw = jnp.concatenate([hd['scaling'][0], hd['rotation'][0],
                          hd['opacity'][0], hd['shs'][0], hd['xyz'][0]],
                         axis=1)
    hb = jnp.concatenate([hd['scaling'][1], hd['rotation'][1],
                          hd['opacity'][1], hd['shs'][1], hd['xyz'][1]])
    wts = [pp2['bn'][0].reshape(1, -1), pp2['bn'][1].reshape(1, -1),
           pp2['f2'][0], pp2['f2'][1].reshape(1, -1),
           d1w[:256], d1w[256:], d1b.reshape(1, -1), d2w,
           d2b.reshape(1, -1), hw, hb.reshape(1, -1)]

    def body(h2_r, ssum_r, ssq_r, x_r, pc_r,
             bnw_r, bnb_r, f2w_r, f2b_r,
             dg_r, dx_r, d1b_r, d2w_r, d2b_r, hw_r, hb_r,
             xyz_r, op_r, rot_r, sc_r, shs_r):
        g2out = _bn_f2(h2_r[...], ssum_r, ssq_r, bnw_r, bnb_r, f2w_r, f2b_r)
        z = _leaky(_mm(g2out, dg_r[...]) + _mm(x_r[...], dx_r[...])
                   + d1b_r[...], 0.01)
        z = _leaky(_mm(z, d2w_r[...]) + d2b_r[...], 0.01)
        ho = _mm(z, hw_r[...]) + hb_r[...]
        sc = ho[:, 0:3]
        sc_r[...] = jnp.maximum(sc, 0.0) + jnp.log1p(jnp.exp(-jnp.abs(sc)))
        rot = ho[:, 3:7]
        nrm = jnp.sqrt(jnp.sum(rot * rot, axis=1, keepdims=True))
        rot_r[...] = rot / jnp.maximum(nrm, 1e-12)
        op_r[...] = 1.0 / (1.0 + jnp.exp(-ho[:, 7:8]))
        shs_r[...] = ho[:, 8:11]
        xyz_r[...] = (1.0 / (1.0 + jnp.exp(-ho[:, 11:14])) - 0.5) \
            * (1.2 / 32.0) + pc_r[...]

    return pl.pallas_call(
        body, grid=(GRID,),
        in_specs=[_rowblk(256), _STAT, _STAT, _rowblk(256), _rowblk(3)]
                 + [_full(w) for w in wts],
        out_specs=[_rowblk(3), _rowblk(1), _rowblk(4), _rowblk(3),
                   _rowblk(3)],
        out_shape=[jax.ShapeDtypeStruct((N, 3), _f32),
                   jax.ShapeDtypeStruct((N, 1), _f32),
                   jax.ShapeDtypeStruct((N, 4), _f32),
                   jax.ShapeDtypeStruct((N, 3), _f32),
                   jax.ShapeDtypeStruct((N, 3), _f32)],
    )(h2, ssum2, ssq2, xcat, pc, *wts)


def kernel(pos, params, edge_index, batch):
    p = params
    src = edge_index[0].reshape(E // CH, CH)
    dst = edge_index[1].reshape(E // CH, CH)

    posones = jnp.concatenate(
        [pos, jnp.ones((N, 1), _f32), jnp.zeros((N, 60), _f32)], axis=1)
    we1 = p['linkx1']['edge'][0]
    we2 = p['linkx2']['edge'][0]
    w1c = tuple(we1[:, i * 64:(i + 1) * 64] for i in range(4))
    w2c = tuple(we2[:, i * 64:(i + 1) * 64] for i in range(4))

    x1a, x1b = _rff(pos, p['enc_b'])
    s10, s11, sp, w10, w11, w12, w13 = _edge_segsum(
        src, dst, (x1a, x1b, posones) + w1c, None)
    x2a, x2b = _conv(x1a, x1b, pos, sp, s10, s11, p['conv1'], False)
    s20, s21, w20, w21, w22, w23 = _edge_segsum(
        src, dst, (x2a, x2b) + w2c, None)
    x3a, x3b, mx = _conv(x2a, x2b, pos, sp, s20, s21, p['conv2'], True)
    pc, xcat, h1, ssum, ssq = _tail_linkx1(
        x3a, x3b, mx, (w10, w11, w12, w13), p)
    h2, ssum2, ssq2 = _linkx12(h1, ssum, ssq, (w20, w21, w22, w23), p)
    xyz, opacity, rot, scaling, shs = _linkx2_heads(
        h2, ssum2, ssq2, xcat, pc, p)
    return (xyz, opacity, rot, scaling, shs.reshape(N, 1, 3))


# speedup vs baseline: 1.0366x; 1.0366x over previous
import functools

import jax
import jax.numpy as jnp
from jax import lax
from jax.experimental import pallas as pl
from jax.experimental.pallas import tpu as pltpu
from jax.experimental.pallas import tpu_sc as plsc

N = 16384
E = 262144
BLK = 1024
GRID = N // BLK
CH = 64
NBUF = 8
NSUB = 16
NCORE = 2
NW = NSUB * NCORE
EPW = E // NW
NCH = EPW // CH
RPW = N // NSUB
ZR = 128

_f32 = jnp.float32


W64 = 64


@functools.lru_cache(maxsize=None)
def _make_segsum(T):
    mesh = plsc.VectorSubcoreMesh(core_axis_name="c", subcore_axis_name="s")
    out_type = tuple(jax.ShapeDtypeStruct((2 * N, W64), _f32)
                     for _ in range(T))
    scratch = [pltpu.VMEM((NCH, CH), jnp.int32),
               pltpu.VMEM((NCH, CH), jnp.int32)]
    scratch += [pltpu.VMEM((CH, W64), _f32)] * NBUF
    scratch += [pltpu.VMEM((ZR, W64), _f32),
                pltpu.VMEM_SHARED((N, W64), _f32)]
    scratch += [pltpu.SemaphoreType.DMA] * NBUF

    def body(src_r, dst_r, *rest):
        tabs = rest[:T]
        outs = rest[T:2 * T]
        sidx, didx = rest[2 * T], rest[2 * T + 1]
        bufs = rest[2 * T + 2:2 * T + 2 + NBUF]
        zbuf, acc = rest[2 * T + 2 + NBUF], rest[2 * T + 3 + NBUF]
        sems = rest[2 * T + 4 + NBUF:]

        c = lax.axis_index("c")
        s = lax.axis_index("s")
        wid = c * NSUB + s
        base_r = s * RPW

        pltpu.sync_copy(src_r.at[pl.ds(wid * NCH, NCH)], sidx)
        pltpu.sync_copy(dst_r.at[pl.ds(wid * NCH, NCH)], didx)

        def zinit(i, _):
            zbuf[i // 4, pl.ds((i % 4) * 16, 16)] = jnp.zeros((16,), _f32)
            return 0

        lax.fori_loop(0, ZR * 4, zinit, 0)
        for r0 in range(0, RPW, ZR):
            pltpu.sync_copy(zbuf, acc.at[pl.ds(base_r + r0, ZR)])
        plsc.subcore_barrier()

        for t in range(T):
            tab, out = tabs[t], outs[t]

            def start(i, b, tab=tab):
                pltpu.async_copy(tab.at[sidx.at[i]], bufs[b], sems[b])

            def drain_scatter(i, b, tab=tab):
                pltpu.make_async_copy(tab.at[sidx.at[i]], bufs[b],
                                      sems[b]).wait()
                pltpu.sync_copy(bufs[b], acc.at[didx.at[i]], add=True)

            for k in range(NBUF - 1):
                start(k, k)

            def ring(g, _):
                i = NBUF * g
                for k in range(NBUF):
                    drain_scatter(i + k, k)
                    start(i + k + NBUF - 1, (k + NBUF - 1) % NBUF)
                return 0

            lax.fori_loop(0, NCH // NBUF - 1, ring, 0)
            i = NCH - NBUF
            start(NCH - 1, NBUF - 1)
            for k in range(NBUF):
                drain_scatter(i + k, k)
            plsc.subcore_barrier()

            pltpu.sync_copy(acc.at[pl.ds(base_r, RPW)],
                            out.at[pl.ds(c * N + base_r, RPW)])
            if t + 1 < T:
                for r0 in range(0, RPW, ZR):
                    pltpu.sync_copy(zbuf, acc.at[pl.ds(base_r + r0, ZR)])
            plsc.subcore_barrier()

    return pl.kernel(body, out_type=out_type, mesh=mesh,
                     scratch_types=scratch,
                     compiler_params=pltpu.CompilerParams(
                         use_tc_tiling_on_sc=False))


def _edge_segsum(src, dst, tables, widths):
    del widths
    return _make_segsum(len(tables))(src, dst, *tables)


def _mm(x, w):
    return lax.dot_general(x, w, (((1,), (0,)), ((), ())),
                           preferred_element_type=_f32)


def _full(a):
    return pl.BlockSpec(a.shape, lambda i: (0,) * a.ndim)


def _rowblk(w, half=0):
    return pl.BlockSpec((BLK, w), lambda i, h=half: (i + h * GRID, 0))


def _leaky(x, s):
    return jnp.where(x >= 0, x, s * x)


def _rff(pos, encb):
    def body(pos_r, encb_r, xa_r, xb_r):
        vp = (2.0 * jnp.pi) * lax.dot_general(
            pos_r[...], encb_r[...], (((1,), (1,)), ((), ())),
            preferred_element_type=_f32)
        xa_r[...] = jnp.cos(vp)
        xb_r[...] = jnp.sin(vp)

    return pl.pallas_call(
        body, grid=(GRID,),
        in_specs=[_rowblk(3), _full(encb)],
        out_specs=[_rowblk(64), _rowblk(64)],
        out_shape=[jax.ShapeDtypeStruct((N, 64), _f32)] * 2,
    )(pos, encb)


def _conv(xa, xb, pos, sp, s0, s1, pp, with_max):
    h1w, h1b = pp['h1']
    h2w, h2b = pp['h2']
    g1w, g1b = pp['g1']
    g2w, g2b = pp['g2']
    g1pw, g1xw = g1w[:3], g1w[3:]
    wts = [h1w, h1b.reshape(1, -1), h2w, h2b.reshape(1, -1),
           g1pw, g1xw, g1b.reshape(1, -1), g2w, g2b.reshape(1, -1)]

    def body(xa_r, xb_r, pos_r, spa_r, spb_r, s0a_r, s0b_r, s1a_r, s1b_r,
             h1w_r, h1b_r, h2w_r, h2b_r, g1pw_r, g1xw_r, g1b_r, g2w_r,
             g2b_r, ya_r, yb_r, *mx_r):
        x = jnp.concatenate([xa_r[...], xb_r[...]], axis=1)
        t = jnp.maximum(_mm(x, h1w_r[...]) + h1b_r[...], 0.0)
        delta = jnp.tanh(_mm(t, h2w_r[...]) + h2b_r[...])
        spv = spa_r[...] + spb_r[...]
        apos = spv[:, 0:3]
        cnt = spv[:, 3:4]
        sx = jnp.concatenate([s0a_r[...] + s0b_r[...],
                              s1a_r[...] + s1b_r[...]], axis=1)
        inv = 1.0 / jnp.maximum(cnt, 1.0)
        mpos = (apos - cnt * pos_r[...] + cnt * delta) * inv
        mfeat = sx * inv
        o = jnp.maximum(_mm(mpos, g1pw_r[...]) + _mm(mfeat, g1xw_r[...])
                        + g1b_r[...], 0.0)
        o = jnp.maximum(_mm(o, g2w_r[...]) + g2b_r[...], 0.0)
        y = x + o
        ya_r[...] = y[:, :64]
        yb_r[...] = y[:, 64:]
        if mx_r:
            @pl.when(pl.program_id(0) == 0)
            def _():
                mx_r[0][...] = jnp.full((1, 128), -jnp.inf, _f32)
            mx_r[0][...] = jnp.maximum(mx_r[0][...],
                                       jnp.max(y, axis=0, keepdims=True))

    out_specs = [_rowblk(64), _rowblk(64)]
    out_shape = [jax.ShapeDtypeStruct((N, 64), _f32)] * 2
    if with_max:
        out_specs.append(pl.BlockSpec((1, 128), lambda i: (0, 0)))
        out_shape.append(jax.ShapeDtypeStruct((1, 128), _f32))
    return pl.pallas_call(
        body, grid=(GRID,),
        in_specs=[_rowblk(64), _rowblk(64), _rowblk(3),
                  _rowblk(64), _rowblk(64, 1),
                  _rowblk(64), _rowblk(64, 1), _rowblk(64), _rowblk(64, 1)]
                 + [_full(w) for w in wts],
        out_specs=out_specs, out_shape=out_shape,
    )(xa, xb, pos, sp, sp, s0, s0, s1, s1, *wts)


def _linkx_front(g, xm_src, pp, c1w_r, c1b_r, ndw_r, ndb_r, c2w_r, c2b_r,
                 f1w_r, f1b_r):
    out = g + _mm(g, c1w_r[...]) + c1b_r[...]
    xm = _mm(xm_src, ndw_r[...]) + ndb_r[...]
    out = out + xm + _mm(xm, c2w_r[...]) + c2b_r[...]
    out = jnp.maximum(out, 0.0)
    return jnp.maximum(_mm(out, f1w_r[...]) + f1b_r[...], 0.0)


def _bn_f2(h1, ssum_r, ssq_r, bnw_r, bnb_r, f2w_r, f2b_r):
    mu = ssum_r[...] * (1.0 / N)
    var = ssq_r[...] * (1.0 / N) - mu * mu
    scale = bnw_r[...] / jnp.sqrt(var + 1e-5)
    shift = bnb_r[...] - mu * scale
    return _mm(h1 * scale + shift, f2w_r[...]) + f2b_r[...]


def _stats(h1, ssum_r, ssq_r):
    @pl.when(pl.program_id(0) == 0)
    def _():
        ssum_r[...] = jnp.zeros((1, 256), _f32)
        ssq_r[...] = jnp.zeros((1, 256), _f32)
    ssum_r[...] += jnp.sum(h1, axis=0, keepdims=True)
    ssq_r[...] += jnp.sum(h1 * h1, axis=0, keepdims=True)


def _linkx_wts(pp):
    return [pp['edge'][1].reshape(1, -1), pp['cat1'][0],
            pp['cat1'][1].reshape(1, -1), pp['node'][0],
            pp['node'][1].reshape(1, -1), pp['cat2'][0],
            pp['cat2'][1].reshape(1, -1), pp['f1'][0],
            pp['f1'][1].reshape(1, -1)]


def _gcat(g0a, g0b, g1a, g1b, g2a, g2b, g3a, g3b, eb_r):
    return jnp.concatenate([g0a[...] + g0b[...], g1a[...] + g1b[...],
                            g2a[...] + g2b[...], g3a[...] + g3b[...]],
                           axis=1) + eb_r[...]


def _gspecs():
    return [_rowblk(64), _rowblk(64, 1)] * 4


_STAT = pl.BlockSpec((1, 256), lambda i: (0, 0))


def _tail_linkx1(xa, xb, mx, gparts, params):
    globw, globb = params['glob']
    t1w, t1b = params['tail1']
    t2w, t2b = params['tail2']
    wts = [globw, globb.reshape(1, -1), t1w[:128], t1w[128:],
           t1b.reshape(1, -1), t2w, t2b.reshape(1, -1)] \
        + _linkx_wts(params['linkx1'])

    def body(xa_r, xb_r, mx_r, g0a, g0b, g1a, g1b, g2a, g2b, g3a, g3b,
             gw_r, gb_r, t1x_r, t1h_r, t1b_r, t2w_r, t2b_r,
             eb_r, c1w_r, c1b_r, ndw_r, ndb_r, c2w_r, c2b_r, f1w_r, f1b_r,
             pc_r, xcat_r, h1_r, ssum_r, ssq_r):
        h = _leaky(_mm(mx_r[...], gw_r[...]) + gb_r[...], 0.2)
        x3 = jnp.concatenate([xa_r[...], xb_r[...]], axis=1)
        t = _leaky(_mm(x3, t1x_r[...]) + _mm(h, t1h_r[...]) + t1b_r[...], 0.2)
        pc_r[...] = jnp.tanh(_mm(t, t2w_r[...]) + t2b_r[...])
        xcat = jnp.concatenate([x3, jnp.broadcast_to(h, (BLK, 128))], axis=1)
        xcat_r[...] = xcat
        g = _gcat(g0a, g0b, g1a, g1b, g2a, g2b, g3a, g3b, eb_r)
        h1 = _linkx_front(g, xcat, None, c1w_r, c1b_r, ndw_r, ndb_r,
                          c2w_r, c2b_r, f1w_r, f1b_r)
        h1_r[...] = h1
        _stats(h1, ssum_r, ssq_r)

    return pl.pallas_call(
        body, grid=(GRID,),
        in_specs=[_rowblk(64), _rowblk(64),
                  pl.BlockSpec((1, 128), lambda i: (0, 0))] + _gspecs()
                 + [_full(w) for w in wts],
        out_specs=[_rowblk(3), _rowblk(256), _rowblk(256), _STAT, _STAT],
        out_shape=[jax.ShapeDtypeStruct((N, 3), _f32),
                   jax.ShapeDtypeStruct((N, 256), _f32),
                   jax.ShapeDtypeStruct((N, 256), _f32),
                   jax.ShapeDtypeStruct((1, 256), _f32),
                   jax.ShapeDtypeStruct((1, 256), _f32)],
    )(xa, xb, mx, gparts[0], gparts[0], gparts[1], gparts[1], gparts[2],
      gparts[2], gparts[3], gparts[3], *wts)


def _linkx12(h1, ssum, ssq, gparts, params):
    pp1 = params['linkx1']
    wts = [pp1['bn'][0].reshape(1, -1), pp1['bn'][1].reshape(1, -1),
           pp1['f2'][0], pp1['f2'][1].reshape(1, -1)] \
        + _linkx_wts(params['linkx2'])

    def body(h1_r, ssum_r, ssq_r, g0a, g0b, g1a, g1b, g2a, g2b, g3a, g3b,
             bnw_r, bnb_r, f2w_r, f2b_r,
             eb_r, c1w_r, c1b_r, ndw_r, ndb_r, c2w_r, c2b_r, f1w_r, f1b_r,
             h2_r, ssum2_r, ssq2_r):
        g1out = _bn_f2(h1_r[...], ssum_r, ssq_r, bnw_r, bnb_r, f2w_r, f2b_r)
        g = _gcat(g0a, g0b, g1a, g1b, g2a, g2b, g3a, g3b, eb_r)
        h2 = _linkx_front(g, g1out, None, c1w_r, c1b_r, ndw_r, ndb_r,
                          c2w_r, c2b_r, f1w_r, f1b_r)
        h2_r[...] = h2
        _stats(h2, ssum2_r, ssq2_r)

    return pl.pallas_call(
        body, grid=(GRID,),
        in_specs=[_rowblk(256), _STAT, _STAT] + _gspecs()
                 + [_full(w) for w in wts],
        out_specs=[_rowblk(256), _STAT, _STAT],
        out_shape=[jax.ShapeDtypeStruct((N, 256), _f32),
                   jax.ShapeDtypeStruct((1, 256), _f32),
                   jax.ShapeDtypeStruct((1, 256), _f32)],
    )(h1, ssum, ssq, gparts[0], gparts[0], gparts[1], gparts[1],
      gparts[2], gparts[2], gparts[3], gparts[3], *wts)


def _linkx2_heads(h2, ssum2, ssq2, xcat, pc, params):
    pp2 = params['linkx2']
    d1w, d1b = params['dec1']
    d2w, d2b = params['dec2']
    hd = params['heads']
    hw = jnp.concatenate([hd['scaling'][0], hd['rotation'][0],
                          hd['opacity'][0], hd['shs'][0], hd['xyz'][0]],
                         axis=1)
    hb = jnp.concatenate([hd['scaling'][1], hd['rotation'][1],
                          hd['opacity'][1], hd['shs'][1], hd['xyz'][1]])
    wts = [pp2['bn'][0].reshape(1, -1), pp2['bn'][1].reshape(1, -1),
           pp2['f2'][0], pp2['f2'][1].reshape(1, -1),
           d1w[:256], d1w[256:], d1b.reshape(1, -1), d2w,
           d2b.reshape(1, -1), hw, hb.reshape(1, -1)]

    def body(h2_r, ssum_r, ssq_r, x_r, pc_r,
             bnw_r, bnb_r, f2w_r, f2b_r,
             dg_r, dx_r, d1b_r, d2w_r, d2b_r, hw_r, hb_r,
             xyz_r, op_r, rot_r, sc_r, shs_r):
        g2out = _bn_f2(h2_r[...], ssum_r, ssq_r, bnw_r, bnb_r, f2w_r, f2b_r)
        z = _leaky(_mm(g2out, dg_r[...]) + _mm(x_r[...], dx_r[...])
                   + d1b_r[...], 0.01)
        z = _leaky(_mm(z, d2w_r[...]) + d2b_r[...], 0.01)
        ho = _mm(z, hw_r[...]) + hb_r[...]
        sc = ho[:, 0:3]
        sc_r[...] = jnp.maximum(sc, 0.0) + jnp.log1p(jnp.exp(-jnp.abs(sc)))
        rot = ho[:, 3:7]
        nrm = jnp.sqrt(jnp.sum(rot * rot, axis=1, keepdims=True))
        rot_r[...] = rot / jnp.maximum(nrm, 1e-12)
        op_r[...] = 1.0 / (1.0 + jnp.exp(-ho[:, 7:8]))
        shs_r[...] = ho[:, 8:11]
        xyz_r[...] = (1.0 / (1.0 + jnp.exp(-ho[:, 11:14])) - 0.5) \
            * (1.2 / 32.0) + pc_r[...]

    return pl.pallas_call(
        body, grid=(GRID,),
        in_specs=[_rowblk(256), _STAT, _STAT, _rowblk(256), _rowblk(3)]
                 + [_full(w) for w in wts],
        out_specs=[_rowblk(3), _rowblk(1), _rowblk(4), _rowblk(3),
                   _rowblk(3)],
        out_shape=[jax.ShapeDtypeStruct((N, 3), _f32),
                   jax.ShapeDtypeStruct((N, 1), _f32),
                   jax.ShapeDtypeStruct((N, 4), _f32),
                   jax.ShapeDtypeStruct((N, 3), _f32),
                   jax.ShapeDtypeStruct((N, 3), _f32)],
    )(h2, ssum2, ssq2, xcat, pc, *wts)


def kernel(pos, params, edge_index, batch):
    p = params
    src = edge_index[0].reshape(E // CH, CH)
    dst = edge_index[1].reshape(E // CH, CH)

    posones = jnp.concatenate(
        [pos, jnp.ones((N, 1), _f32), jnp.zeros((N, 60), _f32)], axis=1)
    we1 = p['linkx1']['edge'][0]
    we2 = p['linkx2']['edge'][0]
    wchunks = [we1[:, i * 64:(i + 1) * 64] for i in range(4)] \
        + [we2[:, i * 64:(i + 1) * 64] for i in range(4)]

    sp, w10, w11, w12, w13, w20, w21, w22, w23 = _edge_segsum(
        src, dst, (posones,) + tuple(wchunks), None)

    x1a, x1b = _rff(pos, p['enc_b'])
    s10, s11 = _edge_segsum(src, dst, (x1a, x1b), None)
    x2a, x2b = _conv(x1a, x1b, pos, sp, s10, s11, p['conv1'], False)
    s20, s21 = _edge_segsum(src, dst, (x2a, x2b), None)
    x3a, x3b, mx = _conv(x2a, x2b, pos, sp, s20, s21, p['conv2'], True)
    pc, xcat, h1, ssum, ssq = _tail_linkx1(
        x3a, x3b, mx, (w10, w11, w12, w13), p)
    h2, ssum2, ssq2 = _linkx12(h1, ssum, ssq, (w20, w21, w22, w23), p)
    xyz, opacity, rot, scaling, shs = _linkx2_heads(
        h2, ssum2, ssq2, xcat, pc, p)
    return (xyz, opacity, rot, scaling, shs.reshape(N, 1, 3))


# speedup vs baseline: 1.1170x; 1.0776x over previous
import functools

import jax
import jax.numpy as jnp
from jax import lax
from jax.experimental import pallas as pl
from jax.experimental.pallas import tpu as pltpu
from jax.experimental.pallas import tpu_sc as plsc

N = 16384
E = 262144
BLK = 1024
GRID = N // BLK
CH = 64
NBUF = 8
NSUB = 16
NCORE = 2
NW = NSUB * NCORE
EPW = E // NW
NCH = EPW // CH
RPW = N // NSUB
ZR = 128

_f32 = jnp.float32


W64 = 64


@functools.lru_cache(maxsize=None)
def _make_segsum(T):
    mesh = plsc.VectorSubcoreMesh(core_axis_name="c", subcore_axis_name="s")
    out_type = tuple(jax.ShapeDtypeStruct((2 * N, W64), _f32)
                     for _ in range(T))
    scratch = [pltpu.VMEM((NCH, CH), jnp.int32),
               pltpu.VMEM((NCH, CH), jnp.int32)]
    scratch += [pltpu.VMEM((CH, W64), _f32)] * NBUF
    scratch += [pltpu.VMEM((ZR, W64), _f32),
                pltpu.VMEM_SHARED((N, W64), _f32)]
    scratch += [pltpu.SemaphoreType.DMA] * NBUF

    def body(src_r, dst_r, *rest):
        tabs = rest[:T]
        outs = rest[T:2 * T]
        sidx, didx = rest[2 * T], rest[2 * T + 1]
        bufs = rest[2 * T + 2:2 * T + 2 + NBUF]
        zbuf, acc = rest[2 * T + 2 + NBUF], rest[2 * T + 3 + NBUF]
        sems = rest[2 * T + 4 + NBUF:]

        c = lax.axis_index("c")
        s = lax.axis_index("s")
        wid = c * NSUB + s
        base_r = s * RPW

        pltpu.sync_copy(src_r.at[pl.ds(wid * NCH, NCH)], sidx)
        pltpu.sync_copy(dst_r.at[pl.ds(wid * NCH, NCH)], didx)

        def zinit(i, _):
            zbuf[i // 4, pl.ds((i % 4) * 16, 16)] = jnp.zeros((16,), _f32)
            return 0

        lax.fori_loop(0, ZR * 4, zinit, 0)
        for r0 in range(0, RPW, ZR):
            pltpu.sync_copy(zbuf, acc.at[pl.ds(base_r + r0, ZR)])
        plsc.subcore_barrier()

        for t in range(T):
            tab, out = tabs[t], outs[t]

            def start(i, b, tab=tab):
                pltpu.async_copy(tab.at[sidx.at[i]], bufs[b], sems[b])

            def drain_scatter(i, b, tab=tab):
                pltpu.make_async_copy(tab.at[sidx.at[i]], bufs[b],
                                      sems[b]).wait()
                pltpu.sync_copy(bufs[b], acc.at[didx.at[i]], add=True)

            for k in range(NBUF - 1):
                start(k, k)

            def ring(g, _):
                i = NBUF * g
                for k in range(NBUF):
                    drain_scatter(i + k, k)
                    start(i + k + NBUF - 1, (k + NBUF - 1) % NBUF)
                return 0

            lax.fori_loop(0, NCH // NBUF - 1, ring, 0)
            i = NCH - NBUF
            start(NCH - 1, NBUF - 1)
            for k in range(NBUF):
                drain_scatter(i + k, k)
            plsc.subcore_barrier()

            pltpu.sync_copy(acc.at[pl.ds(base_r, RPW)],
                            out.at[pl.ds(c * N + base_r, RPW)])
            if t + 1 < T:
                for r0 in range(0, RPW, ZR):
                    pltpu.sync_copy(zbuf, acc.at[pl.ds(base_r + r0, ZR)])
            plsc.subcore_barrier()

    return pl.kernel(body, out_type=out_type, mesh=mesh,
                     scratch_types=scratch,
                     compiler_params=pltpu.CompilerParams(
                         use_tc_tiling_on_sc=False))


def _edge_segsum(src, dst, tables, widths):
    del widths
    return _make_segsum(len(tables))(src, dst, *tables)


def _mm(x, w):
    return lax.dot_general(x, w, (((1,), (0,)), ((), ())),
                           preferred_element_type=_f32)


def _full(a):
    return pl.BlockSpec(a.shape, lambda i: (0,) * a.ndim)


def _rowblk(w, half=0):
    return pl.BlockSpec((BLK, w), lambda i, h=half: (i + h * GRID, 0))


def _leaky(x, s):
    return jnp.where(x >= 0, x, s * x)


def _rff(pos, encb):
    def body(pos_r, encb_r, xa_r, xb_r):
        vp = (2.0 * jnp.pi) * lax.dot_general(
            pos_r[...], encb_r[...], (((1,), (1,)), ((), ())),
            preferred_element_type=_f32)
        xa_r[...] = jnp.cos(vp)
        xb_r[...] = jnp.sin(vp)

    return pl.pallas_call(
        body, grid=(GRID,),
        in_specs=[_rowblk(3), _full(encb)],
        out_specs=[_rowblk(64), _rowblk(64)],
        out_shape=[jax.ShapeDtypeStruct((N, 64), _f32)] * 2,
    )(pos, encb)


def _conv(xa, xb, pos, sp, s0, s1, pp, with_max):
    h1w, h1b = pp['h1']
    h2w, h2b = pp['h2']
    g1w, g1b = pp['g1']
    g2w, g2b = pp['g2']
    g1pw, g1xw = g1w[:3], g1w[3:]
    wts = [h1w, h1b.reshape(1, -1), h2w, h2b.reshape(1, -1),
           g1pw, g1xw, g1b.reshape(1, -1), g2w, g2b.reshape(1, -1)]

    def body(xa_r, xb_r, pos_r, spa_r, spb_r, s0a_r, s0b_r, s1a_r, s1b_r,
             h1w_r, h1b_r, h2w_r, h2b_r, g1pw_r, g1xw_r, g1b_r, g2w_r,
             g2b_r, ya_r, yb_r, *mx_r):
        x = jnp.concatenate([xa_r[...], xb_r[...]], axis=1)
        t = jnp.maximum(_mm(x, h1w_r[...]) + h1b_r[...], 0.0)
        delta = jnp.tanh(_mm(t, h2w_r[...]) + h2b_r[...])
        spv = spa_r[...] + spb_r[...]
        apos = spv[:, 0:3]
        cnt = spv[:, 3:4]
        sx = jnp.concatenate([s0a_r[...] + s0b_r[...],
                              s1a_r[...] + s1b_r[...]], axis=1)
        inv = 1.0 / jnp.maximum(cnt, 1.0)
        mpos = (apos - cnt * pos_r[...] + cnt * delta) * inv
        mfeat = sx * inv
        o = jnp.maximum(_mm(mpos, g1pw_r[...]) + _mm(mfeat, g1xw_r[...])
                        + g1b_r[...], 0.0)
        o = jnp.maximum(_mm(o, g2w_r[...]) + g2b_r[...], 0.0)
        y = x + o
        ya_r[...] = y[:, :64]
        yb_r[...] = y[:, 64:]
        if mx_r:
            @pl.when(pl.program_id(0) == 0)
            def _():
                mx_r[0][...] = jnp.full((1, 128), -jnp.inf, _f32)
            mx_r[0][...] = jnp.maximum(mx_r[0][...],
                                       jnp.max(y, axis=0, keepdims=True))

    out_specs = [_rowblk(64), _rowblk(64)]
    out_shape = [jax.ShapeDtypeStruct((N, 64), _f32)] * 2
    if with_max:
        out_specs.append(pl.BlockSpec((1, 128), lambda i: (0, 0)))
        out_shape.append(jax.ShapeDtypeStruct((1, 128), _f32))
    return pl.pallas_call(
        body, grid=(GRID,),
        in_specs=[_rowblk(64), _rowblk(64), _rowblk(3),
                  _rowblk(64), _rowblk(64, 1),
                  _rowblk(64), _rowblk(64, 1), _rowblk(64), _rowblk(64, 1)]
                 + [_full(w) for w in wts],
        out_specs=out_specs, out_shape=out_shape,
    )(xa, xb, pos, sp, sp, s0, s0, s1, s1, *wts)


def _linkx_front(g, xm_src, pp, c1w_r, c1b_r, ndw_r, ndb_r, c2w_r, c2b_r,
                 f1w_r, f1b_r):
    out = g + _mm(g, c1w_r[...]) + c1b_r[...]
    xm = _mm(xm_src, ndw_r[...]) + ndb_r[...]
    out = out + xm + _mm(xm, c2w_r[...]) + c2b_r[...]
    out = jnp.maximum(out, 0.0)
    return jnp.maximum(_mm(out, f1w_r[...]) + f1b_r[...], 0.0)


def _bn_f2(h1, ssum_r, ssq_r, bnw_r, bnb_r, f2w_r, f2b_r):
    mu = ssum_r[...] * (1.0 / N)
    var = ssq_r[...] * (1.0 / N) - mu * mu
    scale = bnw_r[...] / jnp.sqrt(var + 1e-5)
    shift = bnb_r[...] - mu * scale
    return _mm(h1 * scale + shift, f2w_r[...]) + f2b_r[...]


def _stats(h1, ssum_r, ssq_r):
    @pl.when(pl.program_id(0) == 0)
    def _():
        ssum_r[...] = jnp.zeros((1, 256), _f32)
        ssq_r[...] = jnp.zeros((1, 256), _f32)
    ssum_r[...] += jnp.sum(h1, axis=0, keepdims=True)
    ssq_r[...] += jnp.sum(h1 * h1, axis=0, keepdims=True)


def _linkx_wts(pp):
    return [pp['edge'][1].reshape(1, -1), pp['cat1'][0],
            pp['cat1'][1].reshape(1, -1), pp['node'][0],
            pp['node'][1].reshape(1, -1), pp['cat2'][0],
            pp['cat2'][1].reshape(1, -1), pp['f1'][0],
            pp['f1'][1].reshape(1, -1)]


def _gcat(g0a, g0b, g1a, g1b, g2a, g2b, g3a, g3b, eb_r):
    return jnp.concatenate([g0a[...] + g0b[...], g1a[...] + g1b[...],
                            g2a[...] + g2b[...], g3a[...] + g3b[...]],
                           axis=1) + eb_r[...]


def _gspecs():
    return [_rowblk(64), _rowblk(64, 1)] * 4


_STAT = pl.BlockSpec((1, 256), lambda i: (0, 0))


def _tail_linkx1(xa, xb, mx, gparts, params):
    globw, globb = params['glob']
    t1w, t1b = params['tail1']
    t2w, t2b = params['tail2']
    wts = [globw, globb.reshape(1, -1), t1w[:128], t1w[128:],
           t1b.reshape(1, -1), t2w, t2b.reshape(1, -1)] \
        + _linkx_wts(params['linkx1'])

    def body(xa_r, xb_r, mx_r, g0a, g0b, g1a, g1b, g2a, g2b, g3a, g3b,
             gw_r, gb_r, t1x_r, t1h_r, t1b_r, t2w_r, t2b_r,
             eb_r, c1w_r, c1b_r, ndw_r, ndb_r, c2w_r, c2b_r, f1w_r, f1b_r,
             pc_r, xcat_r, h1_r, ssum_r, ssq_r):
        h = _leaky(_mm(mx_r[...], gw_r[...]) + gb_r[...], 0.2)
        x3 = jnp.concatenate([xa_r[...], xb_r[...]], axis=1)
        t = _leaky(_mm(x3, t1x_r[...]) + _mm(h, t1h_r[...]) + t1b_r[...], 0.2)
        pc_r[...] = jnp.tanh(_mm(t, t2w_r[...]) + t2b_r[...])
        xcat = jnp.concatenate([x3, jnp.broadcast_to(h, (BLK, 128))], axis=1)
        xcat_r[...] = xcat
        g = _gcat(g0a, g0b, g1a, g1b, g2a, g2b, g3a, g3b, eb_r)
        h1 = _linkx_front(g, xcat, None, c1w_r, c1b_r, ndw_r, ndb_r,
                          c2w_r, c2b_r, f1w_r, f1b_r)
        h1_r[...] = h1
        _stats(h1, ssum_r, ssq_r)

    return pl.pallas_call(
        body, grid=(GRID,),
        in_specs=[_rowblk(64), _rowblk(64),
                  pl.BlockSpec((1, 128), lambda i: (0, 0))] + _gspecs()
                 + [_full(w) for w in wts],
        out_specs=[_rowblk(3), _rowblk(256), _rowblk(256), _STAT, _STAT],
        out_shape=[jax.ShapeDtypeStruct((N, 3), _f32),
                   jax.ShapeDtypeStruct((N, 256), _f32),
                   jax.ShapeDtypeStruct((N, 256), _f32),
                   jax.ShapeDtypeStruct((1, 256), _f32),
                   jax.ShapeDtypeStruct((1, 256), _f32)],
    )(xa, xb, mx, gparts[0], gparts[0], gparts[1], gparts[1], gparts[2],
      gparts[2], gparts[3], gparts[3], *wts)


def _linkx12(h1, ssum, ssq, gparts, params):
    pp1 = params['linkx1']
    wts = [pp1['bn'][0].reshape(1, -1), pp1['bn'][1].reshape(1, -1),
           pp1['f2'][0], pp1['f2'][1].reshape(1, -1)] \
        + _linkx_wts(params['linkx2'])

    def body(h1_r, ssum_r, ssq_r, g0a, g0b, g1a, g1b, g2a, g2b, g3a, g3b,
             bnw_r, bnb_r, f2w_r, f2b_r,
             eb_r, c1w_r, c1b_r, ndw_r, ndb_r, c2w_r, c2b_r, f1w_r, f1b_r,
             h2_r, ssum2_r, ssq2_r):
        g1out = _bn_f2(h1_r[...], ssum_r, ssq_r, bnw_r, bnb_r, f2w_r, f2b_r)
        g = _gcat(g0a, g0b, g1a, g1b, g2a, g2b, g3a, g3b, eb_r)
        h2 = _linkx_front(g, g1out, None, c1w_r, c1b_r, ndw_r, ndb_r,
                          c2w_r, c2b_r, f1w_r, f1b_r)
        h2_r[...] = h2
        _stats(h2, ssum2_r, ssq2_r)

    return pl.pallas_call(
        body, grid=(GRID,),
        in_specs=[_rowblk(256), _STAT, _STAT] + _gspecs()
                 + [_full(w) for w in wts],
        out_specs=[_rowblk(256), _STAT, _STAT],
        out_shape=[jax.ShapeDtypeStruct((N, 256), _f32),
                   jax.ShapeDtypeStruct((1, 256), _f32),
                   jax.ShapeDtypeStruct((1, 256), _f32)],
    )(h1, ssum, ssq, gparts[0], gparts[0], gparts[1], gparts[1],
      gparts[2], gparts[2], gparts[3], gparts[3], *wts)


def _linkx2_heads(h2, ssum2, ssq2, xcat, pc, params):
    pp2 = params['linkx2']
    d1w, d1b = params['dec1']
    d2w, d2b = params['dec2']
    hd = params['heads']
    hw = jnp.concatenate([hd['scaling'][0], hd['rotation'][0],
                          hd['opacity'][0], hd['shs'][0], hd['xyz'][0]],
                         axis=1)
    hb = jnp.concatenate([hd['scaling'][1], hd['rotation'][1],
                          hd['opacity'][1], hd['shs'][1], hd['xyz'][1]])
    wts = [pp2['bn'][0].reshape(1, -1), pp2['bn'][1].reshape(1, -1),
           pp2['f2'][0], pp2['f2'][1].reshape(1, -1),
           d1w[:256], d1w[256:], d1b.reshape(1, -1), d2w,
           d2b.reshape(1, -1), hw, hb.reshape(1, -1)]

    def body(h2_r, ssum_r, ssq_r, x_r, pc_r,
             bnw_r, bnb_r, f2w_r, f2b_r,
             dg_r, dx_r, d1b_r, d2w_r, d2b_r, hw_r, hb_r,
             xyz_r, op_r, rot_r, sc_r, shs_r):
        g2out = _bn_f2(h2_r[...], ssum_r, ssq_r, bnw_r, bnb_r, f2w_r, f2b_r)
        z = _leaky(_mm(g2out, dg_r[...]) + _mm(x_r[...], dx_r[...])
                   + d1b_r[...], 0.01)
        z = _leaky(_mm(z, d2w_r[...]) + d2b_r[...], 0.01)
        ho = _mm(z, hw_r[...]) + hb_r[...]
        sc = ho[:, 0:3]
        sc_r[...] = jnp.maximum(sc, 0.0) + jnp.log1p(jnp.exp(-jnp.abs(sc)))
        rot = ho[:, 3:7]
        nrm = jnp.sqrt(jnp.sum(rot * rot, axis=1, keepdims=True))
        rot_r[...] = rot / jnp.maximum(nrm, 1e-12)
        op_r[...] = 1.0 / (1.0 + jnp.exp(-ho[:, 7:8]))
        shs_r[...] = ho[:, 8:11]
        xyz_r[...] = (1.0 / (1.0 + jnp.exp(-ho[:, 11:14])) - 0.5) \
            * (1.2 / 32.0) + pc_r[...]

    return pl.pallas_call(
        body, grid=(GRID,),
        in_specs=[_rowblk(256), _STAT, _STAT, _rowblk(256), _rowblk(3)]
                 + [_full(w) for w in wts],
        out_specs=[_rowblk(3), _rowblk(1), _rowblk(4), _rowblk(3),
                   _rowblk(3)],
        out_shape=[jax.ShapeDtypeStruct((N, 3), _f32),
                   jax.ShapeDtypeStruct((N, 1), _f32),
                   jax.ShapeDtypeStruct((N, 4), _f32),
                   jax.ShapeDtypeStruct((N, 3), _f32),
                   jax.ShapeDtypeStruct((N, 3), _f32)],
    )(h2, ssum2, ssq2, xcat, pc, *wts)


def kernel(pos, params, edge_index, batch):
    p = params
    src = edge_index[0].reshape(E // CH, CH)
    dst = edge_index[1].reshape(E // CH, CH)

    posones = jnp.concatenate(
        [pos, jnp.ones((N, 1), _f32), jnp.zeros((N, 60), _f32)], axis=1)
    we1 = p['linkx1']['edge'][0]
    we2 = p['linkx2']['edge'][0]
    wchunks = [we1[:, i * 64:(i + 1) * 64] for i in range(4)] \
        + [we2[:, i * 64:(i + 1) * 64] for i in range(4)]

    w10, w11, w12, w13, w20, w21, w22, w23 = _edge_segsum(
        src, dst, tuple(wchunks), None)

    x1a, x1b = _rff(pos, p['enc_b'])
    s10, s11, sp = _edge_segsum(src, dst, (x1a, x1b, posones), None)
    x2a, x2b = _conv(x1a, x1b, pos, sp, s10, s11, p['conv1'], False)
    s20, s21 = _edge_segsum(src, dst, (x2a, x2b), None)
    x3a, x3b, mx = _conv(x2a, x2b, pos, sp, s20, s21, p['conv2'], True)
    pc, xcat, h1, ssum, ssq = _tail_linkx1(
        x3a, x3b, mx, (w10, w11, w12, w13), p)
    h2, ssum2, ssq2 = _linkx12(h1, ssum, ssq, (w20, w21, w22, w23), p)
    xyz, opacity, rot, scaling, shs = _linkx2_heads(
        h2, ssum2, ssq2, xcat, pc, p)
    return (xyz, opacity, rot, scaling, shs.reshape(N, 1, 3))


# speedup vs baseline: 1.1421x; 1.0225x over previous
import functools

import jax
import jax.numpy as jnp
from jax import lax
from jax.experimental import pallas as pl
from jax.experimental.pallas import tpu as pltpu
from jax.experimental.pallas import tpu_sc as plsc

N = 16384
E = 262144
BLK = 1024
GRID = N // BLK
CH = 64
NBUF = 8
NSUB = 16
NCORE = 2
NW = NSUB * NCORE
EPW = E // NW
NCH = EPW // CH
RPW = N // NSUB
ZR = 128

_f32 = jnp.float32


W64 = 64


@functools.lru_cache(maxsize=None)
def _make_segsum(T, W=W64):
    mesh = plsc.VectorSubcoreMesh(core_axis_name="c", subcore_axis_name="s")
    out_type = tuple(jax.ShapeDtypeStruct((2 * N, W), _f32)
                     for _ in range(T))
    scratch = [pltpu.VMEM((NCH, CH), jnp.int32),
               pltpu.VMEM((NCH, CH), jnp.int32)]
    scratch += [pltpu.VMEM((CH, W), _f32)] * NBUF
    scratch += [pltpu.VMEM((ZR, W), _f32),
                pltpu.VMEM_SHARED((N, W), _f32)]
    scratch += [pltpu.SemaphoreType.DMA] * NBUF

    def body(src_r, dst_r, *rest):
        tabs = rest[:T]
        outs = rest[T:2 * T]
        sidx, didx = rest[2 * T], rest[2 * T + 1]
        bufs = rest[2 * T + 2:2 * T + 2 + NBUF]
        zbuf, acc = rest[2 * T + 2 + NBUF], rest[2 * T + 3 + NBUF]
        sems = rest[2 * T + 4 + NBUF:]

        c = lax.axis_index("c")
        s = lax.axis_index("s")
        wid = c * NSUB + s
        base_r = s * RPW

        pltpu.sync_copy(src_r.at[pl.ds(wid * NCH, NCH)], sidx)
        pltpu.sync_copy(dst_r.at[pl.ds(wid * NCH, NCH)], didx)

        nv = W // 16

        def zinit(i, _):
            zbuf[i // nv, pl.ds((i % nv) * 16, 16)] = jnp.zeros((16,), _f32)
            return 0

        lax.fori_loop(0, ZR * nv, zinit, 0)
        for r0 in range(0, RPW, ZR):
            pltpu.sync_copy(zbuf, acc.at[pl.ds(base_r + r0, ZR)])
        plsc.subcore_barrier()

        for t in range(T):
            tab, out = tabs[t], outs[t]

            def start(i, b, tab=tab):
                pltpu.async_copy(tab.at[sidx.at[i]], bufs[b], sems[b])

            def drain_scatter(i, b, tab=tab):
                pltpu.make_async_copy(tab.at[sidx.at[i]], bufs[b],
                                      sems[b]).wait()
                pltpu.sync_copy(bufs[b], acc.at[didx.at[i]], add=True)

            for k in range(NBUF - 1):
                start(k, k)

            def ring(g, _):
                i = NBUF * g
                for k in range(NBUF):
                    drain_scatter(i + k, k)
                    start(i + k + NBUF - 1, (k + NBUF - 1) % NBUF)
                return 0

            lax.fori_loop(0, NCH // NBUF - 1, ring, 0)
            i = NCH - NBUF
            start(NCH - 1, NBUF - 1)
            for k in range(NBUF):
                drain_scatter(i + k, k)
            plsc.subcore_barrier()

            pltpu.sync_copy(acc.at[pl.ds(base_r, RPW)],
                            out.at[pl.ds(c * N + base_r, RPW)])
            if t + 1 < T:
                for r0 in range(0, RPW, ZR):
                    pltpu.sync_copy(zbuf, acc.at[pl.ds(base_r + r0, ZR)])
            plsc.subcore_barrier()

    return pl.kernel(body, out_type=out_type, mesh=mesh,
                     scratch_types=scratch,
                     compiler_params=pltpu.CompilerParams(
                         use_tc_tiling_on_sc=False))


def _edge_segsum(src, dst, tables, widths):
    del widths
    w = tables[0].shape[1]
    return _make_segsum(len(tables), w)(src, dst, *tables)


def _mm(x, w):
    return lax.dot_general(x, w, (((1,), (0,)), ((), ())),
                           preferred_element_type=_f32)


def _full(a):
    return pl.BlockSpec(a.shape, lambda i: (0,) * a.ndim)


def _rowblk(w, half=0):
    return pl.BlockSpec((BLK, w), lambda i, h=half: (i + h * GRID, 0))


def _leaky(x, s):
    return jnp.where(x >= 0, x, s * x)


def _rff(pos, encb):
    def body(pos_r, encb_r, xa_r, xb_r):
        vp = (2.0 * jnp.pi) * lax.dot_general(
            pos_r[...], encb_r[...], (((1,), (1,)), ((), ())),
            preferred_element_type=_f32)
        xa_r[...] = jnp.cos(vp)
        xb_r[...] = jnp.sin(vp)

    return pl.pallas_call(
        body, grid=(GRID,),
        in_specs=[_rowblk(3), _full(encb)],
        out_specs=[_rowblk(64), _rowblk(64)],
        out_shape=[jax.ShapeDtypeStruct((N, 64), _f32)] * 2,
    )(pos, encb)


def _conv(xa, xb, pos, sp, s0, s1, pp, with_max):
    h1w, h1b = pp['h1']
    h2w, h2b = pp['h2']
    g1w, g1b = pp['g1']
    g2w, g2b = pp['g2']
    g1pw, g1xw = g1w[:3], g1w[3:]
    wts = [h1w, h1b.reshape(1, -1), h2w, h2b.reshape(1, -1),
           g1pw, g1xw, g1b.reshape(1, -1), g2w, g2b.reshape(1, -1)]

    def body(xa_r, xb_r, pos_r, spa_r, spb_r, s0a_r, s0b_r, s1a_r, s1b_r,
             h1w_r, h1b_r, h2w_r, h2b_r, g1pw_r, g1xw_r, g1b_r, g2w_r,
             g2b_r, ya_r, yb_r, *mx_r):
        x = jnp.concatenate([xa_r[...], xb_r[...]], axis=1)
        t = jnp.maximum(_mm(x, h1w_r[...]) + h1b_r[...], 0.0)
        delta = jnp.tanh(_mm(t, h2w_r[...]) + h2b_r[...])
        spv = spa_r[...] + spb_r[...]
        apos = spv[:, 0:3]
        cnt = spv[:, 3:4]
        sx = jnp.concatenate([s0a_r[...] + s0b_r[...],
                              s1a_r[...] + s1b_r[...]], axis=1)
        inv = 1.0 / jnp.maximum(cnt, 1.0)
        mpos = (apos - cnt * pos_r[...] + cnt * delta) * inv
        mfeat = sx * inv
        o = jnp.maximum(_mm(mpos, g1pw_r[...]) + _mm(mfeat, g1xw_r[...])
                        + g1b_r[...], 0.0)
        o = jnp.maximum(_mm(o, g2w_r[...]) + g2b_r[...], 0.0)
        y = x + o
        ya_r[...] = y[:, :64]
        yb_r[...] = y[:, 64:]
        if mx_r:
            @pl.when(pl.program_id(0) == 0)
            def _():
                mx_r[0][...] = jnp.full((1, 128), -jnp.inf, _f32)
            mx_r[0][...] = jnp.maximum(mx_r[0][...],
                                       jnp.max(y, axis=0, keepdims=True))

    out_specs = [_rowblk(64), _rowblk(64)]
    out_shape = [jax.ShapeDtypeStruct((N, 64), _f32)] * 2
    if with_max:
        out_specs.append(pl.BlockSpec((1, 128), lambda i: (0, 0)))
        out_shape.append(jax.ShapeDtypeStruct((1, 128), _f32))
    return pl.pallas_call(
        body, grid=(GRID,),
        in_specs=[_rowblk(64), _rowblk(64), _rowblk(3),
                  _rowblk(16), _rowblk(16, 1),
                  _rowblk(64), _rowblk(64, 1), _rowblk(64), _rowblk(64, 1)]
                 + [_full(w) for w in wts],
        out_specs=out_specs, out_shape=out_shape,
    )(xa, xb, pos, sp, sp, s0, s0, s1, s1, *wts)


def _linkx_front(g, xm_src, pp, c1w_r, c1b_r, ndw_r, ndb_r, c2w_r, c2b_r,
                 f1w_r, f1b_r):
    out = g + _mm(g, c1w_r[...]) + c1b_r[...]
    xm = _mm(xm_src, ndw_r[...]) + ndb_r[...]
    out = out + xm + _mm(xm, c2w_r[...]) + c2b_r[...]
    out = jnp.maximum(out, 0.0)
    return jnp.maximum(_mm(out, f1w_r[...]) + f1b_r[...], 0.0)


def _bn_f2(h1, ssum_r, ssq_r, bnw_r, bnb_r, f2w_r, f2b_r):
    mu = ssum_r[...] * (1.0 / N)
    var = ssq_r[...] * (1.0 / N) - mu * mu
    scale = bnw_r[...] / jnp.sqrt(var + 1e-5)
    shift = bnb_r[...] - mu * scale
    return _mm(h1 * scale + shift, f2w_r[...]) + f2b_r[...]


def _stats(h1, ssum_r, ssq_r):
    @pl.when(pl.program_id(0) == 0)
    def _():
        ssum_r[...] = jnp.zeros((1, 256), _f32)
        ssq_r[...] = jnp.zeros((1, 256), _f32)
    ssum_r[...] += jnp.sum(h1, axis=0, keepdims=True)
    ssq_r[...] += jnp.sum(h1 * h1, axis=0, keepdims=True)


def _linkx_wts(pp):
    return [pp['edge'][1].reshape(1, -1), pp['cat1'][0],
            pp['cat1'][1].reshape(1, -1), pp['node'][0],
            pp['node'][1].reshape(1, -1), pp['cat2'][0],
            pp['cat2'][1].reshape(1, -1), pp['f1'][0],
            pp['f1'][1].reshape(1, -1)]


def _gcat(g0a, g0b, g1a, g1b, g2a, g2b, g3a, g3b, eb_r):
    return jnp.concatenate([g0a[...] + g0b[...], g1a[...] + g1b[...],
                            g2a[...] + g2b[...], g3a[...] + g3b[...]],
                           axis=1) + eb_r[...]


def _gspecs():
    return [_rowblk(64), _rowblk(64, 1)] * 4


_STAT = pl.BlockSpec((1, 256), lambda i: (0, 0))


def _tail_linkx1(xa, xb, mx, gparts, params):
    globw, globb = params['glob']
    t1w, t1b = params['tail1']
    t2w, t2b = params['tail2']
    wts = [globw, globb.reshape(1, -1), t1w[:128], t1w[128:],
           t1b.reshape(1, -1), t2w, t2b.reshape(1, -1)] \
        + _linkx_wts(params['linkx1'])

    def body(xa_r, xb_r, mx_r, g0a, g0b, g1a, g1b, g2a, g2b, g3a, g3b,
             gw_r, gb_r, t1x_r, t1h_r, t1b_r, t2w_r, t2b_r,
             eb_r, c1w_r, c1b_r, ndw_r, ndb_r, c2w_r, c2b_r, f1w_r, f1b_r,
             pc_r, xcat_r, h1_r, ssum_r, ssq_r):
        h = _leaky(_mm(mx_r[...], gw_r[...]) + gb_r[...], 0.2)
        x3 = jnp.concatenate([xa_r[...], xb_r[...]], axis=1)
        t = _leaky(_mm(x3, t1x_r[...]) + _mm(h, t1h_r[...]) + t1b_r[...], 0.2)
        pc_r[...] = jnp.tanh(_mm(t, t2w_r[...]) + t2b_r[...])
        xcat = jnp.concatenate([x3, jnp.broadcast_to(h, (BLK, 128))], axis=1)
        xcat_r[...] = xcat
        g = _gcat(g0a, g0b, g1a, g1b, g2a, g2b, g3a, g3b, eb_r)
        h1 = _linkx_front(g, xcat, None, c1w_r, c1b_r, ndw_r, ndb_r,
                          c2w_r, c2b_r, f1w_r, f1b_r)
        h1_r[...] = h1
        _stats(h1, ssum_r, ssq_r)

    return pl.pallas_call(
        body, grid=(GRID,),
        in_specs=[_rowblk(64), _rowblk(64),
                  pl.BlockSpec((1, 128), lambda i: (0, 0))] + _gspecs()
                 + [_full(w) for w in wts],
        out_specs=[_rowblk(3), _rowblk(256), _rowblk(256), _STAT, _STAT],
        out_shape=[jax.ShapeDtypeStruct((N, 3), _f32),
                   jax.ShapeDtypeStruct((N, 256), _f32),
                   jax.ShapeDtypeStruct((N, 256), _f32),
                   jax.ShapeDtypeStruct((1, 256), _f32),
                   jax.ShapeDtypeStruct((1, 256), _f32)],
    )(xa, xb, mx, gparts[0], gparts[0], gparts[1], gparts[1], gparts[2],
      gparts[2], gparts[3], gparts[3], *wts)


def _linkx12(h1, ssum, ssq, gparts, params):
    pp1 = params['linkx1']
    wts = [pp1['bn'][0].reshape(1, -1), pp1['bn'][1].reshape(1, -1),
           pp1['f2'][0], pp1['f2'][1].reshape(1, -1)] \
        + _linkx_wts(params['linkx2'])

    def body(h1_r, ssum_r, ssq_r, g0a, g0b, g1a, g1b, g2a, g2b, g3a, g3b,
             bnw_r, bnb_r, f2w_r, f2b_r,
             eb_r, c1w_r, c1b_r, ndw_r, ndb_r, c2w_r, c2b_r, f1w_r, f1b_r,
             h2_r, ssum2_r, ssq2_r):
        g1out = _bn_f2(h1_r[...], ssum_r, ssq_r, bnw_r, bnb_r, f2w_r, f2b_r)
        g = _gcat(g0a, g0b, g1a, g1b, g2a, g2b, g3a, g3b, eb_r)
        h2 = _linkx_front(g, g1out, None, c1w_r, c1b_r, ndw_r, ndb_r,
                          c2w_r, c2b_r, f1w_r, f1b_r)
        h2_r[...] = h2
        _stats(h2, ssum2_r, ssq2_r)

    return pl.pallas_call(
        body, grid=(GRID,),
        in_specs=[_rowblk(256), _STAT, _STAT] + _gspecs()
                 + [_full(w) for w in wts],
        out_specs=[_rowblk(256), _STAT, _STAT],
        out_shape=[jax.ShapeDtypeStruct((N, 256), _f32),
                   jax.ShapeDtypeStruct((1, 256), _f32),
                   jax.ShapeDtypeStruct((1, 256), _f32)],
    )(h1, ssum, ssq, gparts[0], gparts[0], gparts[1], gparts[1],
      gparts[2], gparts[2], gparts[3], gparts[3], *wts)


def _linkx2_heads(h2, ssum2, ssq2, xcat, pc, params):
    pp2 = params['linkx2']
    d1w, d1b = params['dec1']
    d2w, d2b = params['dec2']
    hd = params['heads']
    hw = jnp.concatenate([hd['scaling'][0], hd['rotation'][0],
                          hd['opacity'][0], hd['shs'][0], hd['xyz'][0]],
                         axis=1)
    hb = jnp.concatenate([hd['scaling'][1], hd['rotation'][1],
                          hd['opacity'][1], hd['shs'][1], hd['xyz'][1]])
    wts = [pp2['bn'][0].reshape(1, -1), pp2['bn'][1].reshape(1, -1),
           pp2['f2'][0], pp2['f2'][1].reshape(1, -1),
           d1w[:256], d1w[256:], d1b.reshape(1, -1), d2w,
           d2b.reshape(1, -1), hw, hb.reshape(1, -1)]

    def body(h2_r, ssum_r, ssq_r, x_r, pc_r,
             bnw_r, bnb_r, f2w_r, f2b_r,
             dg_r, dx_r, d1b_r, d2w_r, d2b_r, hw_r, hb_r,
             xyz_r, op_r, rot_r, sc_r, shs_r):
        g2out = _bn_f2(h2_r[...], ssum_r, ssq_r, bnw_r, bnb_r, f2w_r, f2b_r)
        z = _leaky(_mm(g2out, dg_r[...]) + _mm(x_r[...], dx_r[...])
                   + d1b_r[...], 0.01)
        z = _leaky(_mm(z, d2w_r[...]) + d2b_r[...], 0.01)
        ho = _mm(z, hw_r[...]) + hb_r[...]
        sc = ho[:, 0:3]
        sc_r[...] = jnp.maximum(sc, 0.0) + jnp.log1p(jnp.exp(-jnp.abs(sc)))
        rot = ho[:, 3:7]
        nrm = jnp.sqrt(jnp.sum(rot * rot, axis=1, keepdims=True))
        rot_r[...] = rot / jnp.maximum(nrm, 1e-12)
        op_r[...] = 1.0 / (1.0 + jnp.exp(-ho[:, 7:8]))
        shs_r[...] = ho[:, 8:11]
        xyz_r[...] = (1.0 / (1.0 + jnp.exp(-ho[:, 11:14])) - 0.5) \
            * (1.2 / 32.0) + pc_r[...]

    return pl.pallas_call(
        body, grid=(GRID,),
        in_specs=[_rowblk(256), _STAT, _STAT, _rowblk(256), _rowblk(3)]
                 + [_full(w) for w in wts],
        out_specs=[_rowblk(3), _rowblk(1), _rowblk(4), _rowblk(3),
                   _rowblk(3)],
        out_shape=[jax.ShapeDtypeStruct((N, 3), _f32),
                   jax.ShapeDtypeStruct((N, 1), _f32),
                   jax.ShapeDtypeStruct((N, 4), _f32),
                   jax.ShapeDtypeStruct((N, 3), _f32),
                   jax.ShapeDtypeStruct((N, 3), _f32)],
    )(h2, ssum2, ssq2, xcat, pc, *wts)


def kernel(pos, params, edge_index, batch):
    p = params
    src = edge_index[0].reshape(E // CH, CH)
    dst = edge_index[1].reshape(E // CH, CH)

    posones = jnp.concatenate(
        [pos, jnp.ones((N, 1), _f32), jnp.zeros((N, 12), _f32)], axis=1)
    we1 = p['linkx1']['edge'][0]
    we2 = p['linkx2']['edge'][0]
    wchunks = [we1[:, i * 64:(i + 1) * 64] for i in range(4)] \
        + [we2[:, i * 64:(i + 1) * 64] for i in range(4)]

    (sp,) = _edge_segsum(src, dst, (posones,), None)
    w10, w11, w12, w13, w20, w21, w22, w23 = _edge_segsum(
        src, dst, tuple(wchunks), None)

    x1a, x1b = _rff(pos, p['enc_b'])
    s10, s11 = _edge_segsum(src, dst, (x1a, x1b), None)
    x2a, x2b = _conv(x1a, x1b, pos, sp, s10, s11, p['conv1'], False)
    s20, s21 = _edge_segsum(src, dst, (x2a, x2b), None)
    x3a, x3b, mx = _conv(x2a, x2b, pos, sp, s20, s21, p['conv2'], True)
    pc, xcat, h1, ssum, ssq = _tail_linkx1(
        x3a, x3b, mx, (w10, w11, w12, w13), p)
    h2, ssum2, ssq2 = _linkx12(h1, ssum, ssq, (w20, w21, w22, w23), p)
    xyz, opacity, rot, scaling, shs = _linkx2_heads(
        h2, ssum2, ssq2, xcat, pc, p)
    return (xyz, opacity, rot, scaling, shs.reshape(N, 1, 3))


# speedup vs baseline: 1.1560x; 1.0122x over previous
import functools

import jax
import jax.numpy as jnp
from jax import lax
from jax.experimental import pallas as pl
from jax.experimental.pallas import tpu as pltpu
from jax.experimental.pallas import tpu_sc as plsc

N = 16384
E = 262144
BLK = 1024
GRID = N // BLK
CH = 64
NBUF = 8
NSUB = 16
NCORE = 2
NW = NSUB * NCORE
EPW = E // NW
NCH = EPW // CH
RPW = N // NSUB
ZR = 128

_f32 = jnp.float32


W64 = 64


@functools.lru_cache(maxsize=None)
def _make_segsum(T, W=W64):
    mesh = plsc.VectorSubcoreMesh(core_axis_name="c", subcore_axis_name="s")
    out_type = tuple(jax.ShapeDtypeStruct((2 * N, W), _f32)
                     for _ in range(T))
    scratch = [pltpu.VMEM((NCH, CH), jnp.int32),
               pltpu.VMEM((NCH, CH), jnp.int32)]
    scratch += [pltpu.VMEM((CH, W), _f32)] * NBUF
    scratch += [pltpu.VMEM((ZR, W), _f32),
                pltpu.VMEM_SHARED((N, W), _f32)]
    scratch += [pltpu.SemaphoreType.DMA] * NBUF

    def body(src_r, dst_r, *rest):
        tabs = rest[:T]
        outs = rest[T:2 * T]
        sidx, didx = rest[2 * T], rest[2 * T + 1]
        bufs = rest[2 * T + 2:2 * T + 2 + NBUF]
        zbuf, acc = rest[2 * T + 2 + NBUF], rest[2 * T + 3 + NBUF]
        sems = rest[2 * T + 4 + NBUF:]

        c = lax.axis_index("c")
        s = lax.axis_index("s")
        wid = c * NSUB + s
        base_r = s * RPW

        pltpu.sync_copy(src_r.at[pl.ds(wid * NCH, NCH)], sidx)
        pltpu.sync_copy(dst_r.at[pl.ds(wid * NCH, NCH)], didx)

        nv = W // 16

        def zinit(i, _):
            zbuf[i // nv, pl.ds((i % nv) * 16, 16)] = jnp.zeros((16,), _f32)
            return 0

        lax.fori_loop(0, ZR * nv, zinit, 0)
        for r0 in range(0, RPW, ZR):
            pltpu.sync_copy(zbuf, acc.at[pl.ds(base_r + r0, ZR)])
        plsc.subcore_barrier()

        def prime(tab):
            for k in range(NBUF - 1):
                pltpu.async_copy(tab.at[sidx.at[k]], bufs[k], sems[k])

        prime(tabs[0])
        for t in range(T):
            tab, out = tabs[t], outs[t]

            def start(i, b, tab=tab):
                pltpu.async_copy(tab.at[sidx.at[i]], bufs[b], sems[b])

            def drain_scatter(i, b, tab=tab):
                pltpu.make_async_copy(tab.at[sidx.at[i]], bufs[b],
                                      sems[b]).wait()
                pltpu.sync_copy(bufs[b], acc.at[didx.at[i]], add=True)

            def ring(g, _):
                i = NBUF * g
                for k in range(NBUF):
                    drain_scatter(i + k, k)
                    start(i + k + NBUF - 1, (k + NBUF - 1) % NBUF)
                return 0

            lax.fori_loop(0, NCH // NBUF - 1, ring, 0)
            i = NCH - NBUF
            start(NCH - 1, NBUF - 1)
            for k in range(NBUF):
                drain_scatter(i + k, k)
            plsc.subcore_barrier()

            if t + 1 < T:
                prime(tabs[t + 1])
            pltpu.sync_copy(acc.at[pl.ds(base_r, RPW)],
                            out.at[pl.ds(c * N + base_r, RPW)])
            if t + 1 < T:
                for r0 in range(0, RPW, ZR):
                    pltpu.sync_copy(zbuf, acc.at[pl.ds(base_r + r0, ZR)])
            plsc.subcore_barrier()

    return pl.kernel(body, out_type=out_type, mesh=mesh,
                     scratch_types=scratch,
                     compiler_params=pltpu.CompilerParams(
                         use_tc_tiling_on_sc=False))


def _edge_segsum(src, dst, tables, widths):
    del widths
    w = tables[0].shape[1]
    return _make_segsum(len(tables), w)(src, dst, *tables)


def _mm(x, w):
    return lax.dot_general(x, w, (((1,), (0,)), ((), ())),
                           preferred_element_type=_f32)


def _full(a):
    return pl.BlockSpec(a.shape, lambda i: (0,) * a.ndim)


def _rowblk(w, half=0):
    return pl.BlockSpec((BLK, w), lambda i, h=half: (i + h * GRID, 0))


def _leaky(x, s):
    return jnp.where(x >= 0, x, s * x)


def _rff(pos, encb):
    def body(pos_r, encb_r, xa_r, xb_r):
        vp = (2.0 * jnp.pi) * lax.dot_general(
            pos_r[...], encb_r[...], (((1,), (1,)), ((), ())),
            preferred_element_type=_f32)
        xa_r[...] = jnp.cos(vp)
        xb_r[...] = jnp.sin(vp)

    return pl.pallas_call(
        body, grid=(GRID,),
        in_specs=[_rowblk(3), _full(encb)],
        out_specs=[_rowblk(64), _rowblk(64)],
        out_shape=[jax.ShapeDtypeStruct((N, 64), _f32)] * 2,
    )(pos, encb)


def _conv(xa, xb, pos, sp, s0, s1, pp, with_max):
    h1w, h1b = pp['h1']
    h2w, h2b = pp['h2']
    g1w, g1b = pp['g1']
    g2w, g2b = pp['g2']
    g1pw, g1xw = g1w[:3], g1w[3:]
    wts = [h1w, h1b.reshape(1, -1), h2w, h2b.reshape(1, -1),
           g1pw, g1xw, g1b.reshape(1, -1), g2w, g2b.reshape(1, -1)]

    def body(xa_r, xb_r, pos_r, spa_r, spb_r, s0a_r, s0b_r, s1a_r, s1b_r,
             h1w_r, h1b_r, h2w_r, h2b_r, g1pw_r, g1xw_r, g1b_r, g2w_r,
             g2b_r, ya_r, yb_r, *mx_r):
        x = jnp.concatenate([xa_r[...], xb_r[...]], axis=1)
        t = jnp.maximum(_mm(x, h1w_r[...]) + h1b_r[...], 0.0)
        delta = jnp.tanh(_mm(t, h2w_r[...]) + h2b_r[...])
        spv = spa_r[...] + spb_r[...]
        apos = spv[:, 0:3]
        cnt = spv[:, 3:4]
        sx = jnp.concatenate([s0a_r[...] + s0b_r[...],
                              s1a_r[...] + s1b_r[...]], axis=1)
        inv = 1.0 / jnp.maximum(cnt, 1.0)
        mpos = (apos - cnt * pos_r[...] + cnt * delta) * inv
        mfeat = sx * inv
        o = jnp.maximum(_mm(mpos, g1pw_r[...]) + _mm(mfeat, g1xw_r[...])
                        + g1b_r[...], 0.0)
        o = jnp.maximum(_mm(o, g2w_r[...]) + g2b_r[...], 0.0)
        y = x + o
        ya_r[...] = y[:, :64]
        yb_r[...] = y[:, 64:]
        if mx_r:
            @pl.when(pl.program_id(0) == 0)
            def _():
                mx_r[0][...] = jnp.full((1, 128), -jnp.inf, _f32)
            mx_r[0][...] = jnp.maximum(mx_r[0][...],
                                       jnp.max(y, axis=0, keepdims=True))

    out_specs = [_rowblk(64), _rowblk(64)]
    out_shape = [jax.ShapeDtypeStruct((N, 64), _f32)] * 2
    if with_max:
        out_specs.append(pl.BlockSpec((1, 128), lambda i: (0, 0)))
        out_shape.append(jax.ShapeDtypeStruct((1, 128), _f32))
    return pl.pallas_call(
        body, grid=(GRID,),
        in_specs=[_rowblk(64), _rowblk(64), _rowblk(3),
                  _rowblk(16), _rowblk(16, 1),
                  _rowblk(64), _rowblk(64, 1), _rowblk(64), _rowblk(64, 1)]
                 + [_full(w) for w in wts],
        out_specs=out_specs, out_shape=out_shape,
    )(xa, xb, pos, sp, sp, s0, s0, s1, s1, *wts)


def _linkx_front(g, xm_src, pp, c1w_r, c1b_r, ndw_r, ndb_r, c2w_r, c2b_r,
                 f1w_r, f1b_r):
    out = g + _mm(g, c1w_r[...]) + c1b_r[...]
    xm = _mm(xm_src, ndw_r[...]) + ndb_r[...]
    out = out + xm + _mm(xm, c2w_r[...]) + c2b_r[...]
    out = jnp.maximum(out, 0.0)
    return jnp.maximum(_mm(out, f1w_r[...]) + f1b_r[...], 0.0)


def _bn_f2(h1, ssum_r, ssq_r, bnw_r, bnb_r, f2w_r, f2b_r):
    mu = ssum_r[...] * (1.0 / N)
    var = ssq_r[...] * (1.0 / N) - mu * mu
    scale = bnw_r[...] / jnp.sqrt(var + 1e-5)
    shift = bnb_r[...] - mu * scale
    return _mm(h1 * scale + shift, f2w_r[...]) + f2b_r[...]


def _stats(h1, ssum_r, ssq_r):
    @pl.when(pl.program_id(0) == 0)
    def _():
        ssum_r[...] = jnp.zeros((1, 256), _f32)
        ssq_r[...] = jnp.zeros((1, 256), _f32)
    ssum_r[...] += jnp.sum(h1, axis=0, keepdims=True)
    ssq_r[...] += jnp.sum(h1 * h1, axis=0, keepdims=True)


def _linkx_wts(pp):
    return [pp['edge'][1].reshape(1, -1), pp['cat1'][0],
            pp['cat1'][1].reshape(1, -1), pp['node'][0],
            pp['node'][1].reshape(1, -1), pp['cat2'][0],
            pp['cat2'][1].reshape(1, -1), pp['f1'][0],
            pp['f1'][1].reshape(1, -1)]


def _gcat(g0a, g0b, g1a, g1b, g2a, g2b, g3a, g3b, eb_r):
    return jnp.concatenate([g0a[...] + g0b[...], g1a[...] + g1b[...],
                            g2a[...] + g2b[...], g3a[...] + g3b[...]],
                           axis=1) + eb_r[...]


def _gspecs():
    return [_rowblk(64), _rowblk(64, 1)] * 4


_STAT = pl.BlockSpec((1, 256), lambda i: (0, 0))


def _tail_linkx1(xa, xb, mx, gparts, params):
    globw, globb = params['glob']
    t1w, t1b = params['tail1']
    t2w, t2b = params['tail2']
    wts = [globw, globb.reshape(1, -1), t1w[:128], t1w[128:],
           t1b.reshape(1, -1), t2w, t2b.reshape(1, -1)] \
        + _linkx_wts(params['linkx1'])

    def body(xa_r, xb_r, mx_r, g0a, g0b, g1a, g1b, g2a, g2b, g3a, g3b,
             gw_r, gb_r, t1x_r, t1h_r, t1b_r, t2w_r, t2b_r,
             eb_r, c1w_r, c1b_r, ndw_r, ndb_r, c2w_r, c2b_r, f1w_r, f1b_r,
             pc_r, xcat_r, h1_r, ssum_r, ssq_r):
        h = _leaky(_mm(mx_r[...], gw_r[...]) + gb_r[...], 0.2)
        x3 = jnp.concatenate([xa_r[...], xb_r[...]], axis=1)
        t = _leaky(_mm(x3, t1x_r[...]) + _mm(h, t1h_r[...]) + t1b_r[...], 0.2)
        pc_r[...] = jnp.tanh(_mm(t, t2w_r[...]) + t2b_r[...])
        xcat = jnp.concatenate([x3, jnp.broadcast_to(h, (BLK, 128))], axis=1)
        xcat_r[...] = xcat
        g = _gcat(g0a, g0b, g1a, g1b, g2a, g2b, g3a, g3b, eb_r)
        h1 = _linkx_front(g, xcat, None, c1w_r, c1b_r, ndw_r, ndb_r,
                          c2w_r, c2b_r, f1w_r, f1b_r)
        h1_r[...] = h1
        _stats(h1, ssum_r, ssq_r)

    return pl.pallas_call(
        body, grid=(GRID,),
        in_specs=[_rowblk(64), _rowblk(64),
                  pl.BlockSpec((1, 128), lambda i: (0, 0))] + _gspecs()
                 + [_full(w) for w in wts],
        out_specs=[_rowblk(3), _rowblk(256), _rowblk(256), _STAT, _STAT],
        out_shape=[jax.ShapeDtypeStruct((N, 3), _f32),
                   jax.ShapeDtypeStruct((N, 256), _f32),
                   jax.ShapeDtypeStruct((N, 256), _f32),
                   jax.ShapeDtypeStruct((1, 256), _f32),
                   jax.ShapeDtypeStruct((1, 256), _f32)],
    )(xa, xb, mx, gparts[0], gparts[0], gparts[1], gparts[1], gparts[2],
      gparts[2], gparts[3], gparts[3], *wts)


def _linkx12(h1, ssum, ssq, gparts, params):
    pp1 = params['linkx1']
    wts = [pp1['bn'][0].reshape(1, -1), pp1['bn'][1].reshape(1, -1),
           pp1['f2'][0], pp1['f2'][1].reshape(1, -1)] \
        + _linkx_wts(params['linkx2'])

    def body(h1_r, ssum_r, ssq_r, g0a, g0b, g1a, g1b, g2a, g2b, g3a, g3b,
             bnw_r, bnb_r, f2w_r, f2b_r,
             eb_r, c1w_r, c1b_r, ndw_r, ndb_r, c2w_r, c2b_r, f1w_r, f1b_r,
             h2_r, ssum2_r, ssq2_r):
        g1out = _bn_f2(h1_r[...], ssum_r, ssq_r, bnw_r, bnb_r, f2w_r, f2b_r)
        g = _gcat(g0a, g0b, g1a, g1b, g2a, g2b, g3a, g3b, eb_r)
        h2 = _linkx_front(g, g1out, None, c1w_r, c1b_r, ndw_r, ndb_r,
                          c2w_r, c2b_r, f1w_r, f1b_r)
        h2_r[...] = h2
        _stats(h2, ssum2_r, ssq2_r)

    return pl.pallas_call(
        body, grid=(GRID,),
        in_specs=[_rowblk(256), _STAT, _STAT] + _gspecs()
                 + [_full(w) for w in wts],
        out_specs=[_rowblk(256), _STAT, _STAT],
        out_shape=[jax.ShapeDtypeStruct((N, 256), _f32),
                   jax.ShapeDtypeStruct((1, 256), _f32),
                   jax.ShapeDtypeStruct((1, 256), _f32)],
    )(h1, ssum, ssq, gparts[0], gparts[0], gparts[1], gparts[1],
      gparts[2], gparts[2], gparts[3], gparts[3], *wts)


def _linkx2_heads(h2, ssum2, ssq2, xcat, pc, params):
    pp2 = params['linkx2']
    d1w, d1b = params['dec1']
    d2w, d2b = params['dec2']
    hd = params['heads']
    hw = jnp.concatenate([hd['scaling'][0], hd['rotation'][0],
                          hd['opacity'][0], hd['shs'][0], hd['xyz'][0]],
                         axis=1)
    hb = jnp.concatenate([hd['scaling'][1], hd['rotation'][1],
                          hd['opacity'][1], hd['shs'][1], hd['xyz'][1]])
    wts = [pp2['bn'][0].reshape(1, -1), pp2['bn'][1].reshape(1, -1),
           pp2['f2'][0], pp2['f2'][1].reshape(1, -1),
           d1w[:256], d1w[256:], d1b.reshape(1, -1), d2w,
           d2b.reshape(1, -1), hw, hb.reshape(1, -1)]

    def body(h2_r, ssum_r, ssq_r, x_r, pc_r,
             bnw_r, bnb_r, f2w_r, f2b_r,
             dg_r, dx_r, d1b_r, d2w_r, d2b_r, hw_r, hb_r,
             xyz_r, op_r, rot_r, sc_r, shs_r):
        g2out = _bn_f2(h2_r[...], ssum_r, ssq_r, bnw_r, bnb_r, f2w_r, f2b_r)
        z = _leaky(_mm(g2out, dg_r[...]) + _mm(x_r[...], dx_r[...])
                   + d1b_r[...], 0.01)
        z = _leaky(_mm(z, d2w_r[...]) + d2b_r[...], 0.01)
        ho = _mm(z, hw_r[...]) + hb_r[...]
        sc = ho[:, 0:3]
        sc_r[...] = jnp.maximum(sc, 0.0) + jnp.log1p(jnp.exp(-jnp.abs(sc)))
        rot = ho[:, 3:7]
        nrm = jnp.sqrt(jnp.sum(rot * rot, axis=1, keepdims=True))
        rot_r[...] = rot / jnp.maximum(nrm, 1e-12)
        op_r[...] = 1.0 / (1.0 + jnp.exp(-ho[:, 7:8]))
        shs_r[...] = ho[:, 8:11]
        xyz_r[...] = (1.0 / (1.0 + jnp.exp(-ho[:, 11:14])) - 0.5) \
            * (1.2 / 32.0) + pc_r[...]

    return pl.pallas_call(
        body, grid=(GRID,),
        in_specs=[_rowblk(256), _STAT, _STAT, _rowblk(256), _rowblk(3)]
                 + [_full(w) for w in wts],
        out_specs=[_rowblk(3), _rowblk(1), _rowblk(4), _rowblk(3),
                   _rowblk(3)],
        out_shape=[jax.ShapeDtypeStruct((N, 3), _f32),
                   jax.ShapeDtypeStruct((N, 1), _f32),
                   jax.ShapeDtypeStruct((N, 4), _f32),
                   jax.ShapeDtypeStruct((N, 3), _f32),
                   jax.ShapeDtypeStruct((N, 3), _f32)],
    )(h2, ssum2, ssq2, xcat, pc, *wts)


def kernel(pos, params, edge_index, batch):
    p = params
    src = edge_index[0].reshape(E // CH, CH)
    dst = edge_index[1].reshape(E // CH, CH)

    posones = jnp.concatenate(
        [pos, jnp.ones((N, 1), _f32), jnp.zeros((N, 12), _f32)], axis=1)
    we1 = p['linkx1']['edge'][0]
    we2 = p['linkx2']['edge'][0]
    wchunks = [we1[:, i * 64:(i + 1) * 64] for i in range(4)] \
        + [we2[:, i * 64:(i + 1) * 64] for i in range(4)]

    (sp,) = _edge_segsum(src, dst, (posones,), None)
    w10, w11, w12, w13, w20, w21, w22, w23 = _edge_segsum(
        src, dst, tuple(wchunks), None)

    x1a, x1b = _rff(pos, p['enc_b'])
    s10, s11 = _edge_segsum(src, dst, (x1a, x1b), None)
    x2a, x2b = _conv(x1a, x1b, pos, sp, s10, s11, p['conv1'], False)
    s20, s21 = _edge_segsum(src, dst, (x2a, x2b), None)
    x3a, x3b, mx = _conv(x2a, x2b, pos, sp, s20, s21, p['conv2'], True)
    pc, xcat, h1, ssum, ssq = _tail_linkx1(
        x3a, x3b, mx, (w10, w11, w12, w13), p)
    h2, ssum2, ssq2 = _linkx12(h1, ssum, ssq, (w20, w21, w22, w23), p)
    xyz, opacity, rot, scaling, shs = _linkx2_heads(
        h2, ssum2, ssq2, xcat, pc, p)
    return (xyz, opacity, rot, scaling, shs.reshape(N, 1, 3))
